# Initial kernel scaffold; baseline (speedup 1.0000x reference)
import jax
import jax.numpy as jnp
from jax.experimental import pallas as pl


def kernel(x, edge_index, batch, W0a, b0a, g0a, be0a, W0b, b0b, g0b, be0b, W1a, b1a, g1a, be1a, W1b, b1b, g1b, be1b, W2a, b2a, g2a, be2a, W2b, b2b, g2b, be2b, linW, linb):
    raise NotImplementedError("write your pallas kernel here")


# speedup vs baseline: 6.9463x; 6.9463x over previous
import functools

import jax
import jax.numpy as jnp
from jax import lax
from jax.experimental import pallas as pl
from jax.experimental.pallas import tpu as pltpu
from jax.experimental.pallas import tpu_sc as plsc

_N = 10000
_E = 320000
_D = 128
_G = 128

_NC = 2
_NS = 16
_NW = _NC * _NS
_EW = _E // _NW
_K = 80
_T = _EW // _K
_RPT = 632
_NP = _NS * _RPT

_BLK = 400
_NBLK = _N // _BLK


def _sc_agg_body(x_hbm, src_hbm, dst_hbm, zeros_hbm, out_hbm,
                 acc, si0, si1, di0, di1, r0, r1, gs0, gs1, is0, is1):
    c = lax.axis_index("c")
    s = lax.axis_index("s")
    w = s * _NC + c

    pltpu.sync_copy(zeros_hbm, acc.at[pl.ds(s * _RPT, _RPT)])

    def idx_start(t, si, di, sem):
        pltpu.async_copy(src_hbm.at[w, t], si, sem)
        pltpu.async_copy(dst_hbm.at[w, t], di, sem)

    def idx_wait(t, si, di, sem):
        pltpu.make_async_copy(src_hbm.at[w, t], si, sem).wait()
        pltpu.make_async_copy(dst_hbm.at[w, t], di, sem).wait()

    def gather_start(si, r, sem):
        pltpu.async_copy(x_hbm.at[si], r, sem)

    def gather_wait(si, r, sem):
        pltpu.make_async_copy(x_hbm.at[si], r, sem).wait()

    plsc.subcore_barrier()

    idx_start(0, si0, di0, is0)
    idx_wait(0, si0, di0, is0)
    gather_start(si0, r0, gs0)
    idx_start(1, si1, di1, is1)

    def _pair(i, _):
        t0 = 2 * i
        gather_wait(si0, r0, gs0)
        idx_wait(t0 + 1, si1, di1, is1)
        gather_start(si1, r1, gs1)
        pltpu.sync_copy(r0, acc.at[di0], add=True)
        idx_start(t0 + 2, si0, di0, is0)
        gather_wait(si1, r1, gs1)
        idx_wait(t0 + 2, si0, di0, is0)
        gather_start(si0, r0, gs0)
        pltpu.sync_copy(r1, acc.at[di1], add=True)

        @pl.when(t0 + 3 < _T)
        def _():
            idx_start(t0 + 3, si1, di1, is1)
        return 0
    lax.fori_loop(0, (_T - 1) // 2, _pair, 0)

    gather_wait(si0, r0, gs0)
    pltpu.sync_copy(r0, acc.at[di0], add=True)

    plsc.subcore_barrier()

    pltpu.sync_copy(acc.at[pl.ds(s * _RPT, _RPT)],
                    out_hbm.at[c, pl.ds(s * _RPT, _RPT)])


_sc_agg = functools.partial(
    pl.kernel,
    out_type=jax.ShapeDtypeStruct((_NC, _NP, _D), jnp.float32),
    mesh=plsc.VectorSubcoreMesh(core_axis_name="c", subcore_axis_name="s"),
    scratch_types=[
        pltpu.VMEM_SHARED((_NP, _D), jnp.float32),
        pltpu.VMEM((_K,), jnp.int32),
        pltpu.VMEM((_K,), jnp.int32),
        pltpu.VMEM((_K,), jnp.int32),
        pltpu.VMEM((_K,), jnp.int32),
        pltpu.VMEM((_K, _D), jnp.float32),
        pltpu.VMEM((_K, _D), jnp.float32),
        pltpu.SemaphoreType.DMA,
        pltpu.SemaphoreType.DMA,
        pltpu.SemaphoreType.DMA,
        pltpu.SemaphoreType.DMA,
    ],
)(_sc_agg_body)


def _stats_accum(i, y, st_ref):
    s1 = jnp.sum(y, axis=0, keepdims=True)
    s2 = jnp.sum(y * y, axis=0, keepdims=True)
    upd = jnp.concatenate([s1, s2, jnp.zeros((6, _D), jnp.float32)], axis=0)

    @pl.when(i == 0)
    def _():
        st_ref[...] = upd

    @pl.when(i > 0)
    def _():
        st_ref[...] += upd


def _bn_affine(st, g, be):
    m = st[0:1, :] * (1.0 / _N)
    v = st[1:2, :] * (1.0 / _N) - m * m
    a = lax.rsqrt(v + 1e-5) * g
    return a, be - m * a


def _passA_body(h_ref, a0_ref, a1_ref, w_ref, b_ref, y_ref, st_ref):
    i = pl.program_id(0)
    u = h_ref[...] + a0_ref[0] + a1_ref[0]
    y = jnp.dot(u, w_ref[...], preferred_element_type=jnp.float32) + b_ref[...]
    y_ref[...] = y
    _stats_accum(i, y, st_ref)


def _passB_body(y1_ref, st1_ref, g_ref, be_ref, w_ref, b_ref, y2_ref, st2_ref):
    i = pl.program_id(0)
    a, cc = _bn_affine(st1_ref[...], g_ref[...], be_ref[...])
    z = jnp.maximum(y1_ref[...] * a + cc, 0.0)
    y2 = jnp.dot(z, w_ref[...], preferred_element_type=jnp.float32) + b_ref[...]
    y2_ref[...] = y2
    _stats_accum(i, y2, st2_ref)


def _passC_body(y2_ref, st_ref, g_ref, be_ref, h_ref):
    a, cc = _bn_affine(st_ref[...], g_ref[...], be_ref[...])
    h_ref[...] = jnp.maximum(y2_ref[...] * a + cc, 0.0)


def _passC2_body(y2_ref, st_ref, g_ref, be_ref, bt_ref, lw_ref, lb_ref,
                 out_ref, pacc):
    i = pl.program_id(0)
    a, cc = _bn_affine(st_ref[...], g_ref[...], be_ref[...])
    h = jnp.maximum(y2_ref[...] * a + cc, 0.0)
    bt = bt_ref[0, 0, :]
    onehot = (bt[:, None] == lax.broadcasted_iota(jnp.int32, (_BLK, _G), 1)
              ).astype(jnp.float32)
    p = lax.dot_general(onehot, h, (((0,), (0,)), ((), ())),
                        preferred_element_type=jnp.float32)

    @pl.when(i == 0)
    def _():
        pacc[...] = p

    @pl.when(i > 0)
    def _():
        pacc[...] += p

    @pl.when(i == _NBLK - 1)
    def _():
        out_ref[...] = (jnp.dot(pacc[...], lw_ref[...],
                                preferred_element_type=jnp.float32)
                        + lb_ref[...])


_blk2 = pl.BlockSpec((_BLK, _D), lambda i: (i, 0))
_full_st = pl.BlockSpec((8, _D), lambda i: (0, 0))
_full_w = pl.BlockSpec((_D, _D), lambda i: (0, 0))
_full_v = pl.BlockSpec((1, _D), lambda i: (0, 0))

_passA = pl.pallas_call(
    _passA_body,
    grid=(_NBLK,),
    in_specs=[
        _blk2,
        pl.BlockSpec((1, _BLK, _D), lambda i: (0, i, 0)),
        pl.BlockSpec((1, _BLK, _D), lambda i: (1, i, 0)),
        _full_w, _full_v,
    ],
    out_specs=[_blk2, _full_st],
    out_shape=[jax.ShapeDtypeStruct((_N, _D), jnp.float32),
               jax.ShapeDtypeStruct((8, _D), jnp.float32)],
)

_passB = pl.pallas_call(
    _passB_body,
    grid=(_NBLK,),
    in_specs=[_blk2, _full_st, _full_v, _full_v, _full_w, _full_v],
    out_specs=[_blk2, _full_st],
    out_shape=[jax.ShapeDtypeStruct((_N, _D), jnp.float32),
               jax.ShapeDtypeStruct((8, _D), jnp.float32)],
)

_passC = pl.pallas_call(
    _passC_body,
    grid=(_NBLK,),
    in_specs=[_blk2, _full_st, _full_v, _full_v],
    out_specs=_blk2,
    out_shape=jax.ShapeDtypeStruct((_N, _D), jnp.float32),
)

_passC2 = pl.pallas_call(
    _passC2_body,
    grid=(_NBLK,),
    in_specs=[
        _blk2, _full_st, _full_v, _full_v,
        pl.BlockSpec((1, 1, _BLK), lambda i: (i, 0, 0)),
        pl.BlockSpec((_D, 2 * _D), lambda i: (0, 0)),
        pl.BlockSpec((1, 2 * _D), lambda i: (0, 0)),
    ],
    out_specs=pl.BlockSpec((_G, 2 * _D), lambda i: (0, 0)),
    out_shape=jax.ShapeDtypeStruct((_G, 2 * _D), jnp.float32),
    scratch_shapes=[pltpu.VMEM((_G, _D), jnp.float32)],
)


def kernel(x, edge_index, batch,
           W0a, b0a, g0a, be0a, W0b, b0b, g0b, be0b,
           W1a, b1a, g1a, be1a, W1b, b1b, g1b, be1b,
           W2a, b2a, g2a, be2a, W2b, b2b, g2b, be2b,
           linW, linb):
    src3 = edge_index[0].astype(jnp.int32).reshape(_NW, _T, _K)
    dst3 = edge_index[1].astype(jnp.int32).reshape(_NW, _T, _K)
    bt3 = batch.astype(jnp.int32).reshape(_NBLK, 1, _BLK)

    p = {
        "W0a": W0a, "b0a": b0a, "g0a": g0a, "be0a": be0a,
        "W0b": W0b, "b0b": b0b, "g0b": g0b, "be0b": be0b,
        "W1a": W1a, "b1a": b1a, "g1a": g1a, "be1a": be1a,
        "W1b": W1b, "b1b": b1b, "g1b": g1b, "be1b": be1b,
        "W2a": W2a, "b2a": b2a, "g2a": g2a, "be2a": be2a,
        "W2b": W2b, "b2b": b2b, "g2b": g2b, "be2b": be2b,
    }

    def row(v):
        return v.reshape(1, -1)

    zc = jnp.zeros((_RPT, _D), jnp.float32)
    h = x
    for l in range(3):
        agg = _sc_agg(h, src3, dst3, zc)
        y1, s1 = _passA(h, agg, agg, p[f"W{l}a"], row(p[f"b{l}a"]))
        y2, s2 = _passB(y1, s1, row(p[f"g{l}a"]), row(p[f"be{l}a"]),
                        p[f"W{l}b"], row(p[f"b{l}b"]))
        if l < 2:
            h = _passC(y2, s2, row(p[f"g{l}b"]), row(p[f"be{l}b"]))
        else:
            out = _passC2(y2, s2, row(p[f"g{l}b"]), row(p[f"be{l}b"]),
                          bt3, linW, row(linb))
    return out


# speedup vs baseline: 8.0117x; 1.1534x over previous
import functools

import jax
import jax.numpy as jnp
from jax import lax
from jax.experimental import pallas as pl
from jax.experimental.pallas import tpu as pltpu
from jax.experimental.pallas import tpu_sc as plsc

_N = 10000
_E = 320000
_D = 128
_G = 128

_NC = 2
_NS = 16
_NW = _NC * _NS
_EW = _E // _NW
_K = 80
_T = _EW // _K
_RPT = 632
_NP = _NS * _RPT

_BLK = 2000
_NBLK = _N // _BLK


def _sc_agg_body(x_hbm, src_hbm, dst_hbm, zeros_hbm, out_hbm,
                 acc, si0, si1, di0, di1, r0, r1, gs0, gs1, is0, is1):
    c = lax.axis_index("c")
    s = lax.axis_index("s")
    w = s * _NC + c

    pltpu.sync_copy(zeros_hbm, acc.at[pl.ds(s * _RPT, _RPT)])

    def idx_start(t, si, di, sem):
        pltpu.async_copy(src_hbm.at[w, t], si, sem)
        pltpu.async_copy(dst_hbm.at[w, t], di, sem)

    def idx_wait(t, si, di, sem):
        pltpu.make_async_copy(src_hbm.at[w, t], si, sem).wait()
        pltpu.make_async_copy(dst_hbm.at[w, t], di, sem).wait()

    def gather_start(si, r, sem):
        pltpu.async_copy(x_hbm.at[si], r, sem)

    def gather_wait(si, r, sem):
        pltpu.make_async_copy(x_hbm.at[si], r, sem).wait()

    plsc.subcore_barrier()

    idx_start(0, si0, di0, is0)
    idx_wait(0, si0, di0, is0)
    gather_start(si0, r0, gs0)
    idx_start(1, si1, di1, is1)

    def _pair(i, _):
        t0 = 2 * i
        gather_wait(si0, r0, gs0)
        idx_wait(t0 + 1, si1, di1, is1)
        gather_start(si1, r1, gs1)
        pltpu.sync_copy(r0, acc.at[di0], add=True)
        idx_start(t0 + 2, si0, di0, is0)
        gather_wait(si1, r1, gs1)
        idx_wait(t0 + 2, si0, di0, is0)
        gather_start(si0, r0, gs0)
        pltpu.sync_copy(r1, acc.at[di1], add=True)

        @pl.when(t0 + 3 < _T)
        def _():
            idx_start(t0 + 3, si1, di1, is1)
        return 0
    lax.fori_loop(0, (_T - 1) // 2, _pair, 0)

    gather_wait(si0, r0, gs0)
    pltpu.sync_copy(r0, acc.at[di0], add=True)

    plsc.subcore_barrier()

    pltpu.sync_copy(acc.at[pl.ds(s * _RPT, _RPT)],
                    out_hbm.at[c, pl.ds(s * _RPT, _RPT)])


_sc_agg = functools.partial(
    pl.kernel,
    out_type=jax.ShapeDtypeStruct((_NC, _NP, _D), jnp.float32),
    mesh=plsc.VectorSubcoreMesh(core_axis_name="c", subcore_axis_name="s"),
    scratch_types=[
        pltpu.VMEM_SHARED((_NP, _D), jnp.float32),
        pltpu.VMEM((_K,), jnp.int32),
        pltpu.VMEM((_K,), jnp.int32),
        pltpu.VMEM((_K,), jnp.int32),
        pltpu.VMEM((_K,), jnp.int32),
        pltpu.VMEM((_K, _D), jnp.float32),
        pltpu.VMEM((_K, _D), jnp.float32),
        pltpu.SemaphoreType.DMA,
        pltpu.SemaphoreType.DMA,
        pltpu.SemaphoreType.DMA,
        pltpu.SemaphoreType.DMA,
    ],
)(_sc_agg_body)


def _stats_accum(i, y, st_ref):
    s1 = jnp.sum(y, axis=0, keepdims=True)
    s2 = jnp.sum(y * y, axis=0, keepdims=True)
    upd = jnp.concatenate([s1, s2, jnp.zeros((6, _D), jnp.float32)], axis=0)

    @pl.when(i == 0)
    def _():
        st_ref[...] = upd

    @pl.when(i > 0)
    def _():
        st_ref[...] += upd


def _bn_affine(st, g, be):
    m = st[0:1, :] * (1.0 / _N)
    v = st[1:2, :] * (1.0 / _N) - m * m
    a = lax.rsqrt(v + 1e-5) * g
    return a, be - m * a


def _passA_body(h_ref, a0_ref, a1_ref, w_ref, b_ref, y_ref, st_ref):
    i = pl.program_id(0)
    u = h_ref[...] + a0_ref[0] + a1_ref[0]
    y = jnp.dot(u, w_ref[...], preferred_element_type=jnp.float32) + b_ref[...]
    y_ref[...] = y
    _stats_accum(i, y, st_ref)


def _passB_body(y1_ref, st1_ref, g_ref, be_ref, w_ref, b_ref, y2_ref, st2_ref):
    i = pl.program_id(0)
    a, cc = _bn_affine(st1_ref[...], g_ref[...], be_ref[...])
    z = jnp.maximum(y1_ref[...] * a + cc, 0.0)
    y2 = jnp.dot(z, w_ref[...], preferred_element_type=jnp.float32) + b_ref[...]
    y2_ref[...] = y2
    _stats_accum(i, y2, st2_ref)


def _passC_body(y2_ref, st_ref, g_ref, be_ref, h_ref):
    a, cc = _bn_affine(st_ref[...], g_ref[...], be_ref[...])
    h_ref[...] = jnp.maximum(y2_ref[...] * a + cc, 0.0)


def _passC2_body(y2_ref, st_ref, g_ref, be_ref, bt_ref, lw_ref, lb_ref,
                 out_ref, pacc):
    i = pl.program_id(0)
    a, cc = _bn_affine(st_ref[...], g_ref[...], be_ref[...])
    h = jnp.maximum(y2_ref[...] * a + cc, 0.0)
    bt = bt_ref[0, 0, :]
    onehot = (bt[:, None] == lax.broadcasted_iota(jnp.int32, (_BLK, _G), 1)
              ).astype(jnp.float32)
    p = lax.dot_general(onehot, h, (((0,), (0,)), ((), ())),
                        preferred_element_type=jnp.float32)

    @pl.when(i == 0)
    def _():
        pacc[...] = p

    @pl.when(i > 0)
    def _():
        pacc[...] += p

    @pl.when(i == _NBLK - 1)
    def _():
        out_ref[...] = (jnp.dot(pacc[...], lw_ref[...],
                                preferred_element_type=jnp.float32)
                        + lb_ref[...])


_blk2 = pl.BlockSpec((_BLK, _D), lambda i: (i, 0))
_full_st = pl.BlockSpec((8, _D), lambda i: (0, 0))
_full_w = pl.BlockSpec((_D, _D), lambda i: (0, 0))
_full_v = pl.BlockSpec((1, _D), lambda i: (0, 0))

_passA = pl.pallas_call(
    _passA_body,
    grid=(_NBLK,),
    in_specs=[
        _blk2,
        pl.BlockSpec((1, _BLK, _D), lambda i: (0, i, 0)),
        pl.BlockSpec((1, _BLK, _D), lambda i: (1, i, 0)),
        _full_w, _full_v,
    ],
    out_specs=[_blk2, _full_st],
    out_shape=[jax.ShapeDtypeStruct((_N, _D), jnp.float32),
               jax.ShapeDtypeStruct((8, _D), jnp.float32)],
)

_passB = pl.pallas_call(
    _passB_body,
    grid=(_NBLK,),
    in_specs=[_blk2, _full_st, _full_v, _full_v, _full_w, _full_v],
    out_specs=[_blk2, _full_st],
    out_shape=[jax.ShapeDtypeStruct((_N, _D), jnp.float32),
               jax.ShapeDtypeStruct((8, _D), jnp.float32)],
)

_passC = pl.pallas_call(
    _passC_body,
    grid=(_NBLK,),
    in_specs=[_blk2, _full_st, _full_v, _full_v],
    out_specs=_blk2,
    out_shape=jax.ShapeDtypeStruct((_N, _D), jnp.float32),
)

_passC2 = pl.pallas_call(
    _passC2_body,
    grid=(_NBLK,),
    in_specs=[
        _blk2, _full_st, _full_v, _full_v,
        pl.BlockSpec((1, 1, _BLK), lambda i: (i, 0, 0)),
        pl.BlockSpec((_D, 2 * _D), lambda i: (0, 0)),
        pl.BlockSpec((1, 2 * _D), lambda i: (0, 0)),
    ],
    out_specs=pl.BlockSpec((_G, 2 * _D), lambda i: (0, 0)),
    out_shape=jax.ShapeDtypeStruct((_G, 2 * _D), jnp.float32),
    scratch_shapes=[pltpu.VMEM((_G, _D), jnp.float32)],
)


def kernel(x, edge_index, batch,
           W0a, b0a, g0a, be0a, W0b, b0b, g0b, be0b,
           W1a, b1a, g1a, be1a, W1b, b1b, g1b, be1b,
           W2a, b2a, g2a, be2a, W2b, b2b, g2b, be2b,
           linW, linb):
    src3 = edge_index[0].astype(jnp.int32).reshape(_NW, _T, _K)
    dst3 = edge_index[1].astype(jnp.int32).reshape(_NW, _T, _K)
    bt3 = batch.astype(jnp.int32).reshape(_NBLK, 1, _BLK)

    p = {
        "W0a": W0a, "b0a": b0a, "g0a": g0a, "be0a": be0a,
        "W0b": W0b, "b0b": b0b, "g0b": g0b, "be0b": be0b,
        "W1a": W1a, "b1a": b1a, "g1a": g1a, "be1a": be1a,
        "W1b": W1b, "b1b": b1b, "g1b": g1b, "be1b": be1b,
        "W2a": W2a, "b2a": b2a, "g2a": g2a, "be2a": be2a,
        "W2b": W2b, "b2b": b2b, "g2b": g2b, "be2b": be2b,
    }

    def row(v):
        return v.reshape(1, -1)

    zc = jnp.zeros((_RPT, _D), jnp.float32)
    h = x
    for l in range(3):
        agg = _sc_agg(h, src3, dst3, zc)
        y1, s1 = _passA(h, agg, agg, p[f"W{l}a"], row(p[f"b{l}a"]))
        y2, s2 = _passB(y1, s1, row(p[f"g{l}a"]), row(p[f"be{l}a"]),
                        p[f"W{l}b"], row(p[f"b{l}b"]))
        if l < 2:
            h = _passC(y2, s2, row(p[f"g{l}b"]), row(p[f"be{l}b"]))
        else:
            out = _passC2(y2, s2, row(p[f"g{l}b"]), row(p[f"be{l}b"]),
                          bt3, linW, row(linb))
    return out
